# Initial kernel scaffold; baseline (speedup 1.0000x reference)
import functools

import jax
import jax.numpy as jnp
from jax import lax
from jax.experimental import pallas as pl
from jax.experimental.pallas import tpu as pltpu
from jax.experimental.pallas import tpu_sc as plsc

B = 4096
L = 200
D = 128
NC = 2
NS = 16
NW = NC * NS
ROWS_PER_W = B // NW
VL = 16
NJ = D // VL
CH0 = 128
CH1 = L - CH0


def _sc_body(body_hbm, table_hbm, out_hbm, idx_v, rows_a, rows_b, t0_v, out_v,
             sem_a, sem_b):
  wid = lax.axis_index("s") * NC + lax.axis_index("c")
  base = wid * ROWS_PER_W

  pltpu.sync_copy(body_hbm.at[pl.ds(base, ROWS_PER_W)], idx_v)
  pltpu.sync_copy(table_hbm.at[pl.ds(0, 1)], t0_v)
  t0 = [t0_v[0, pl.ds(VL * j, VL)] for j in range(NJ)]
  lane = lax.iota(jnp.int32, VL)

  def start_gather(row, rows_ref, sem):
    pltpu.async_copy(table_hbm.at[idx_v.at[row, pl.ds(0, CH0)]],
                     rows_ref.at[pl.ds(0, CH0)], sem)
    pltpu.async_copy(table_hbm.at[idx_v.at[row, pl.ds(CH0, CH1)]],
                     rows_ref.at[pl.ds(CH0, CH1)], sem)

  def wait_gather(rows_ref, sem):
    pltpu.make_async_copy(table_hbm.at[pl.ds(0, L)], rows_ref, sem).wait()

  def process(row, rows_ref):
    cnt = jnp.zeros((VL,), jnp.float32)
    for c in range(L // VL):
      v = idx_v[row, pl.ds(VL * c, VL)]
      cnt = cnt + jnp.where(v == 0, 1.0, 0.0).astype(jnp.float32)
    v = idx_v[row, pl.ds(L - VL, VL)]
    tail = (v == 0) & (lane >= 2 * VL - L % VL)
    cnt = cnt + jnp.where(tail, 1.0, 0.0).astype(jnp.float32)
    c0 = jnp.sum(cnt)
    denom = jnp.maximum(jnp.float32(L) - c0, jnp.float32(1.0))
    inv = jnp.float32(1.0) / denom

    def accum(l, accs):
      return tuple(a + rows_ref[l, pl.ds(VL * j, VL)]
                   for j, a in enumerate(accs))

    accs = lax.fori_loop(0, L, accum,
                         tuple(jnp.zeros((VL,), jnp.float32)
                               for _ in range(NJ)))
    for j in range(NJ):
      out_v[row, pl.ds(VL * j, VL)] = (accs[j] - c0 * t0[j]) * inv

  start_gather(0, rows_a, sem_a)
  start_gather(1, rows_b, sem_b)

  def pair(i, carry):
    row = 2 * i
    wait_gather(rows_a, sem_a)
    process(row, rows_a)
    start_gather(row + 2, rows_a, sem_a)
    wait_gather(rows_b, sem_b)
    process(row + 1, rows_b)
    start_gather(row + 3, rows_b, sem_b)
    return carry

  lax.fori_loop(0, ROWS_PER_W // 2 - 1, pair, 0)
  wait_gather(rows_a, sem_a)
  process(ROWS_PER_W - 2, rows_a)
  wait_gather(rows_b, sem_b)
  process(ROWS_PER_W - 1, rows_b)

  pltpu.sync_copy(out_v, out_hbm.at[pl.ds(base, ROWS_PER_W)])


def _sc_pool(body, emb_table):
  mesh = plsc.VectorSubcoreMesh(core_axis_name="c", subcore_axis_name="s")
  return pl.kernel(
      _sc_body,
      out_type=jax.ShapeDtypeStruct((B, D), jnp.float32),
      mesh=mesh,
      scratch_types=[
          pltpu.VMEM((ROWS_PER_W, L), jnp.int32),
          pltpu.VMEM((L, D), jnp.float32),
          pltpu.VMEM((L, D), jnp.float32),
          pltpu.VMEM((1, D), jnp.float32),
          pltpu.VMEM((ROWS_PER_W, D), jnp.float32),
          pltpu.SemaphoreType.DMA,
          pltpu.SemaphoreType.DMA,
      ],
  )(body, emb_table)


def _bn_body(x_ref, g_ref, b_ref, o_ref):
  x = x_ref[...]
  mu = jnp.mean(x, axis=0, keepdims=True)
  xc = x - mu
  var = jnp.mean(xc * xc, axis=0, keepdims=True)
  o_ref[...] = g_ref[...] * xc * lax.rsqrt(var + 1e-5) + b_ref[...]


def _bn(pooled, gamma, beta):
  return pl.pallas_call(
      _bn_body,
      out_shape=jax.ShapeDtypeStruct((B, D), jnp.float32),
  )(pooled, gamma.reshape(1, D), beta.reshape(1, D))


def kernel(title, body, emb_table, gamma, beta):
  del title
  pooled = _sc_pool(body.astype(jnp.int32), emb_table)
  return _bn(pooled, gamma, beta)


# speedup vs baseline: 15.7957x; 15.7957x over previous
import functools

import jax
import jax.numpy as jnp
from jax import lax
from jax.experimental import pallas as pl
from jax.experimental.pallas import tpu as pltpu
from jax.experimental.pallas import tpu_sc as plsc

B = 4096
L = 200
D = 128
NC = 2
NS = 16
NW = NC * NS
ROWS_PER_W = B // NW
VL = 16
NJ = D // VL
CH0 = 128
CH1 = L - CH0


def _sc_body(body_hbm, table_hbm, out_hbm, idx_v, rows_a, rows_b, t0_v, out_v,
             sem_a, sem_b):
  wid = lax.axis_index("s") * NC + lax.axis_index("c")
  base = wid * ROWS_PER_W

  pltpu.sync_copy(body_hbm.at[pl.ds(base, ROWS_PER_W)], idx_v)
  pltpu.sync_copy(table_hbm.at[pl.ds(0, 1)], t0_v)
  t0 = [t0_v[0, pl.ds(VL * j, VL)] for j in range(NJ)]
  lane = lax.iota(jnp.int32, VL)

  def start_gather(row, rows_ref, sem):
    pltpu.async_copy(table_hbm.at[idx_v.at[row, pl.ds(0, CH0)]],
                     rows_ref.at[pl.ds(0, CH0)], sem)
    pltpu.async_copy(table_hbm.at[idx_v.at[row, pl.ds(CH0, CH1)]],
                     rows_ref.at[pl.ds(CH0, CH1)], sem)

  def wait_gather(rows_ref, sem):
    pltpu.make_async_copy(table_hbm.at[pl.ds(0, L)], rows_ref, sem).wait()

  lane_hi = jnp.minimum(jnp.maximum(lane - (2 * VL - L % VL - 1), 0), 1)

  def process(row, rows_ref):
    nz = jnp.zeros((VL,), jnp.int32)
    for c in range(L // VL):
      v = idx_v[row, pl.ds(VL * c, VL)]
      nz = nz + jnp.minimum(v, 1)
    v = idx_v[row, pl.ds(L - VL, VL)]
    nz = nz + jnp.minimum(v, 1) * lane_hi
    cnt = nz.astype(jnp.float32)
    for k in (8, 4, 2, 1):
      cnt = cnt + cnt[lane ^ k]
    c0 = jnp.float32(L) - cnt
    denom = jnp.maximum(cnt, jnp.float32(1.0))
    inv = jnp.float32(1.0) / denom

    def accum(l, accs):
      return tuple(a + rows_ref[l, pl.ds(VL * j, VL)]
                   for j, a in enumerate(accs))

    accs = lax.fori_loop(0, L, accum,
                         tuple(jnp.zeros((VL,), jnp.float32)
                               for _ in range(NJ)))
    for j in range(NJ):
      out_v[row, pl.ds(VL * j, VL)] = (accs[j] - c0 * t0[j]) * inv

  start_gather(0, rows_a, sem_a)
  start_gather(1, rows_b, sem_b)

  def pair(i, carry):
    row = 2 * i
    wait_gather(rows_a, sem_a)
    process(row, rows_a)
    start_gather(row + 2, rows_a, sem_a)
    wait_gather(rows_b, sem_b)
    process(row + 1, rows_b)
    start_gather(row + 3, rows_b, sem_b)
    return carry

  lax.fori_loop(0, ROWS_PER_W // 2 - 1, pair, 0)
  wait_gather(rows_a, sem_a)
  process(ROWS_PER_W - 2, rows_a)
  wait_gather(rows_b, sem_b)
  process(ROWS_PER_W - 1, rows_b)

  pltpu.sync_copy(out_v, out_hbm.at[pl.ds(base, ROWS_PER_W)])


def _sc_pool(body, emb_table):
  mesh = plsc.VectorSubcoreMesh(core_axis_name="c", subcore_axis_name="s")
  return pl.kernel(
      _sc_body,
      out_type=jax.ShapeDtypeStruct((B, D), jnp.float32),
      mesh=mesh,
      scratch_types=[
          pltpu.VMEM((ROWS_PER_W, L), jnp.int32),
          pltpu.VMEM((L, D), jnp.float32),
          pltpu.VMEM((L, D), jnp.float32),
          pltpu.VMEM((1, D), jnp.float32),
          pltpu.VMEM((ROWS_PER_W, D), jnp.float32),
          pltpu.SemaphoreType.DMA,
          pltpu.SemaphoreType.DMA,
      ],
  )(body, emb_table)


def _bn_body(x_ref, g_ref, b_ref, o_ref):
  x = x_ref[...]
  mu = jnp.mean(x, axis=0, keepdims=True)
  xc = x - mu
  var = jnp.mean(xc * xc, axis=0, keepdims=True)
  o_ref[...] = g_ref[...] * xc * lax.rsqrt(var + 1e-5) + b_ref[...]


def _bn(pooled, gamma, beta):
  return pl.pallas_call(
      _bn_body,
      out_shape=jax.ShapeDtypeStruct((B, D), jnp.float32),
  )(pooled, gamma.reshape(1, D), beta.reshape(1, D))


def kernel(title, body, emb_table, gamma, beta):
  del title
  pooled = _sc_pool(body.astype(jnp.int32), emb_table)
  return _bn(pooled, gamma, beta)
